# baseline (speedup 1.0000x reference)
import functools

import jax
import jax.numpy as jnp
from jax.experimental import pallas as pl
from jax.experimental.pallas import tpu as pltpu

N = 2048
IN_F = 128
OUT_F = 32
AUG = 64
ALPHA = 0.2
BM = 512

_CONTRACT_LAST = (((1,), (1,)), ((), ()))


def _gat_kernel(x_ref, adj_ref, w_ref, a_ref, out_ref, haug_ref, s1_ref, s2_ref):
    i = pl.program_id(0)

    @pl.when(i == 0)
    def _prologue():
        h = jnp.dot(x_ref[...], w_ref[...],
                    preferred_element_type=jnp.float32,
                    precision=jax.lax.Precision.HIGHEST)
        ones = jnp.ones((N, 1), dtype=jnp.float32)
        zeros = jnp.zeros((N, AUG - OUT_F - 1), dtype=jnp.float32)
        haug_ref[...] = jnp.concatenate([h, ones, zeros], axis=1)
        scale = -1.4426950408889634
        s1_ref[...] = jax.lax.dot_general(
            h, scale * a_ref[0:1, :OUT_F], _CONTRACT_LAST,
            preferred_element_type=jnp.float32,
            precision=jax.lax.Precision.HIGHEST)
        s2_ref[...] = jax.lax.dot_general(
            scale * a_ref[0:1, OUT_F:], h, _CONTRACT_LAST,
            preferred_element_type=jnp.float32,
            precision=jax.lax.Precision.HIGHEST)

    s1b = s1_ref[pl.ds(i * BM, BM), :]
    t = s1b + s2_ref[...]
    arg = jnp.minimum(t, ALPHA * t)
    ee = jnp.where(adj_ref[...] != 0, jnp.exp2(arg), 0.0)
    hp_aug = jnp.dot(ee, haug_ref[...],
                     preferred_element_type=jnp.float32)
    hp = hp_aug[:, :OUT_F] / hp_aug[:, OUT_F:OUT_F + 1]
    out_ref[...] = jnp.where(hp > 0, hp, jnp.exp(hp) - 1.0)


@functools.partial(jax.jit, static_argnames=())
def kernel(input, adj, W, a):
    grid = (N // BM,)
    return pl.pallas_call(
        _gat_kernel,
        grid=grid,
        in_specs=[
            pl.BlockSpec((N, IN_F), lambda i: (0, 0)),
            pl.BlockSpec((BM, N), lambda i: (i, 0)),
            pl.BlockSpec((IN_F, OUT_F), lambda i: (0, 0)),
            pl.BlockSpec((1, 2 * OUT_F), lambda i: (0, 0)),
        ],
        out_specs=pl.BlockSpec((BM, OUT_F), lambda i: (i, 0)),
        out_shape=jax.ShapeDtypeStruct((N, OUT_F), jnp.float32),
        scratch_shapes=[
            pltpu.VMEM((N, AUG), jnp.float32),
            pltpu.VMEM((N, 1), jnp.float32),
            pltpu.VMEM((1, N), jnp.float32),
        ],
        compiler_params=pltpu.CompilerParams(
            dimension_semantics=("arbitrary",),
        ),
    )(input, adj, W, a)


# speedup vs baseline: 1.0497x; 1.0497x over previous
import functools

import jax
import jax.numpy as jnp
from jax.experimental import pallas as pl
from jax.experimental.pallas import tpu as pltpu

N = 2048
IN_F = 128
OUT_F = 32
AUG = 64
ALPHA = 0.2
BM = 1024

_CONTRACT_LAST = (((1,), (1,)), ((), ()))


def _gat_kernel(x_ref, adj_ref, w_ref, a_ref, out_ref, haug_ref, s1_ref, s2_ref):
    i = pl.program_id(0)

    @pl.when(i == 0)
    def _prologue():
        h = jnp.dot(x_ref[...], w_ref[...],
                    preferred_element_type=jnp.float32,
                    precision=jax.lax.Precision.HIGHEST)
        ones = jnp.ones((N, 1), dtype=jnp.float32)
        zeros = jnp.zeros((N, AUG - OUT_F - 1), dtype=jnp.float32)
        haug_ref[...] = jnp.concatenate([h, ones, zeros], axis=1)
        scale = -1.4426950408889634
        s1_ref[...] = jax.lax.dot_general(
            h, scale * a_ref[0:1, :OUT_F], _CONTRACT_LAST,
            preferred_element_type=jnp.float32,
            precision=jax.lax.Precision.HIGHEST)
        s2_ref[...] = jax.lax.dot_general(
            scale * a_ref[0:1, OUT_F:], h, _CONTRACT_LAST,
            preferred_element_type=jnp.float32,
            precision=jax.lax.Precision.HIGHEST)

    s1b = s1_ref[pl.ds(i * BM, BM), :]
    t = s1b + s2_ref[...]
    arg = jnp.minimum(t, ALPHA * t)
    ee = jnp.where(adj_ref[...] != 0, jnp.exp2(arg), 0.0)
    hp_aug = jnp.dot(ee, haug_ref[...],
                     preferred_element_type=jnp.float32)
    hp = hp_aug[:, :OUT_F] / hp_aug[:, OUT_F:OUT_F + 1]
    out_ref[...] = jnp.where(hp > 0, hp, jnp.exp(hp) - 1.0)


@functools.partial(jax.jit, static_argnames=())
def kernel(input, adj, W, a):
    grid = (N // BM,)
    return pl.pallas_call(
        _gat_kernel,
        grid=grid,
        in_specs=[
            pl.BlockSpec((N, IN_F), lambda i: (0, 0)),
            pl.BlockSpec((BM, N), lambda i: (i, 0)),
            pl.BlockSpec((IN_F, OUT_F), lambda i: (0, 0)),
            pl.BlockSpec((1, 2 * OUT_F), lambda i: (0, 0)),
        ],
        out_specs=pl.BlockSpec((BM, OUT_F), lambda i: (i, 0)),
        out_shape=jax.ShapeDtypeStruct((N, OUT_F), jnp.float32),
        scratch_shapes=[
            pltpu.VMEM((N, AUG), jnp.float32),
            pltpu.VMEM((N, 1), jnp.float32),
            pltpu.VMEM((1, N), jnp.float32),
        ],
        compiler_params=pltpu.CompilerParams(
            dimension_semantics=("arbitrary",),
        ),
    )(input, adj, W, a)
